# baseline (speedup 1.0000x reference)
import functools

import jax
import jax.numpy as jnp
from jax import lax
from jax.experimental import pallas as pl
from jax.experimental.pallas import tpu as pltpu, tpu_sc as plsc

_DIM = 240
_NSEG = 112
_N = 100000
_NW = 32
_ROWS_PER_W = _N // _NW
_CHUNK = 125
_NCHUNK = _ROWS_PER_W // _CHUNK


def _sqrt16(s):
    y = plsc.bitcast(jnp.int32(0x5F3759DF) - (plsc.bitcast(s, jnp.int32) >> 1),
                     jnp.float32)
    y = y * (1.5 - 0.5 * s * y * y)
    y = y * (1.5 - 0.5 * s * y * y)
    r = s * y
    return r


def _body(x_hbm, o_hbm, xbuf, obuf):
    wid = lax.axis_index("s") * 2 + lax.axis_index("c")
    base_row = wid * _ROWS_PER_W

    lanes = lax.iota(jnp.int32, 16)
    c3a = 64 + 3 * lanes
    c3b = 64 + 48 + 3 * lanes
    c5 = 160 + 5 * lanes

    @pl.loop(0, _NCHUNK)
    def _chunk(c):
        row0 = base_row + c * _CHUNK
        pltpu.sync_copy(x_hbm.at[pl.ds(row0 * _DIM, _CHUNK * _DIM)], xbuf)

        @pl.loop(0, _CHUNK)
        def _row(r):
            ib = r * _DIM
            ob = r * _NSEG
            for t in range(4):
                v = xbuf[pl.ds(ib + t * 16, 16)]
                obuf[pl.ds(ob + t * 16, 16)] = jnp.abs(v)
            for half, cvec in ((0, c3a), (1, c3b)):
                i0 = cvec + ib
                g0 = plsc.load_gather(xbuf, [i0])
                g1 = plsc.load_gather(xbuf, [i0 + 1])
                g2 = plsc.load_gather(xbuf, [i0 + 2])
                s = g0 * g0 + g1 * g1 + g2 * g2
                obuf[pl.ds(ob + 64 + half * 16, 16)] = _sqrt16(s)
            i0 = c5 + ib
            s = None
            for j in range(5):
                g = plsc.load_gather(xbuf, [i0 + j])
                s = g * g if s is None else s + g * g
            obuf[pl.ds(ob + 96, 16)] = _sqrt16(s)

        pltpu.sync_copy(obuf, o_hbm.at[pl.ds(row0 * _NSEG, _CHUNK * _NSEG)])


def kernel(features):
    size = features.shape[:-1]
    x = features.reshape(-1)
    mesh = plsc.VectorSubcoreMesh(core_axis_name="c", subcore_axis_name="s",
                                  num_cores=2, num_subcores=16)
    out = pl.kernel(
        _body,
        out_type=jax.ShapeDtypeStruct((_N * _NSEG,), jnp.float32),
        mesh=mesh,
        scratch_types=[
            pltpu.VMEM((_CHUNK * _DIM,), jnp.float32),
            pltpu.VMEM((_CHUNK * _NSEG,), jnp.float32),
        ],
        compiler_params=pltpu.CompilerParams(needs_layout_passes=False),
    )(x)
    return out.reshape(size + (_NSEG,))


# speedup vs baseline: 1.3453x; 1.3453x over previous
import functools

import jax
import jax.numpy as jnp
from jax import lax
from jax.experimental import pallas as pl
from jax.experimental.pallas import tpu as pltpu, tpu_sc as plsc

_DIM = 240
_NSEG = 112
_N = 100000
_NW = 32
_ROWS_PER_W = _N // _NW
_CHUNK = 125
_NCHUNK = _ROWS_PER_W // _CHUNK
_CD = _CHUNK * _DIM
_CS = _CHUNK * _NSEG


def _sqrt16(s):
    y = plsc.bitcast(jnp.int32(0x5F3759DF) - (plsc.bitcast(s, jnp.int32) >> 1),
                     jnp.float32)
    hs = 0.5 * s
    y = y * (1.5 - hs * y * y)
    y = y * (1.5 - hs * y * y)
    return s * y


def _body(x_hbm, o_hbm, xbuf, obuf, insem, outsem):
    wid = lax.axis_index("s") * 2 + lax.axis_index("c")
    base_row = wid * _ROWS_PER_W

    lanes = lax.iota(jnp.int32, 16)
    c3 = [64 + 3 * lanes, 64 + 48 + 3 * lanes]
    c5 = 160 + 5 * lanes

    def in_copy(c, slot):
        return pltpu.make_async_copy(
            x_hbm.at[pl.ds((base_row + c * _CHUNK) * _DIM, _CD)],
            xbuf.at[pl.ds(slot * _CD, _CD)],
            insem.at[slot])

    def out_copy(c, slot):
        return pltpu.make_async_copy(
            obuf.at[pl.ds(slot * _CS, _CS)],
            o_hbm.at[pl.ds((base_row + c * _CHUNK) * _NSEG, _CS)],
            outsem.at[slot])

    in_copy(0, 0).start()

    @pl.loop(0, _NCHUNK)
    def _chunk(c):
        slot = lax.rem(c, 2)

        @pl.when(c + 1 < _NCHUNK)
        def _prefetch():
            in_copy(c + 1, 1 - slot).start()

        in_copy(c, slot).wait()

        @pl.when(c >= 2)
        def _drain():
            out_copy(c - 2, slot).wait()

        xoff = slot * _CD
        ooff = slot * _CS

        @plsc.parallel_loop(0, _CHUNK, unroll=4)
        def _row(r):
            ib = xoff + r * _DIM
            ob = ooff + r * _NSEG
            row = xbuf.at[pl.ds(ib, _DIM)]
            for t in range(4):
                obuf[pl.ds(ob + t * 16, 16)] = jnp.abs(row[pl.ds(t * 16, 16)])
            for half in range(2):
                g0 = plsc.load_gather(row, [c3[half]])
                g1 = plsc.load_gather(row, [c3[half] + 1])
                g2 = plsc.load_gather(row, [c3[half] + 2])
                s = g0 * g0 + g1 * g1 + g2 * g2
                obuf[pl.ds(ob + 64 + half * 16, 16)] = _sqrt16(s)
            s = None
            for j in range(5):
                g = plsc.load_gather(row, [c5 + j])
                s = g * g if s is None else s + g * g
            obuf[pl.ds(ob + 96, 16)] = _sqrt16(s)

        out_copy(c, slot).start()

    out_copy(_NCHUNK - 2, lax.rem(jnp.int32(_NCHUNK - 2), 2)).wait()
    out_copy(_NCHUNK - 1, lax.rem(jnp.int32(_NCHUNK - 1), 2)).wait()


def kernel(features):
    size = features.shape[:-1]
    x = features.reshape(-1)
    mesh = plsc.VectorSubcoreMesh(core_axis_name="c", subcore_axis_name="s",
                                  num_cores=2, num_subcores=16)
    out = pl.kernel(
        _body,
        out_type=jax.ShapeDtypeStruct((_N * _NSEG,), jnp.float32),
        mesh=mesh,
        scratch_types=[
            pltpu.VMEM((2 * _CD,), jnp.float32),
            pltpu.VMEM((2 * _CS,), jnp.float32),
            pltpu.SemaphoreType.DMA((2,)),
            pltpu.SemaphoreType.DMA((2,)),
        ],
        compiler_params=pltpu.CompilerParams(needs_layout_passes=False),
    )(x)
    return out.reshape(size + (_NSEG,))


# speedup vs baseline: 4.1740x; 3.1027x over previous
import functools

import jax
import jax.numpy as jnp
from jax import lax
from jax.experimental import pallas as pl
from jax.experimental.pallas import tpu as pltpu, tpu_sc as plsc

_DIM = 240
_NSEG = 112
_N = 100000
_NW = 32
_CHUNK = 128
_NFULL = _N // _CHUNK
_TAIL = _N - _NFULL * _CHUNK
_TAIL_W = _NFULL % _NW


def _sqrt16(s):
    y = plsc.bitcast(jnp.int32(0x5F3759DF) - (plsc.bitcast(s, jnp.int32) >> 1),
                     jnp.float32)
    hs = 0.5 * s
    y = y * (1.5 - hs * y * y)
    y = y * (1.5 - hs * y * y)
    return s * y


def _body(x_hbm, o_hbm, xbuf, obuf, insem, outsem):
    wid = lax.axis_index("s") * 2 + lax.axis_index("c")
    nck = jnp.where(wid <= 12, 25, 24)

    lanes = lax.iota(jnp.int32, 16)
    c3 = [64 + 3 * lanes, 64 + 48 + 3 * lanes]
    c5 = 160 + 5 * lanes

    cident = [lanes + 16 * t for t in range(4)]
    cout = [lanes + 16 * t for t in range(7)]

    def row_norm(xr, orow):
        rv = jnp.full((16,), xr, jnp.int32)
        ov = jnp.full((16,), orow, jnp.int32)
        for t in range(4):
            v = plsc.load_gather(xbuf, [rv, cident[t]])
            plsc.store_scatter(obuf, [ov, cout[t]], jnp.abs(v))
        for half in range(2):
            g0 = plsc.load_gather(xbuf, [rv, c3[half]])
            g1 = plsc.load_gather(xbuf, [rv, c3[half] + 1])
            g2 = plsc.load_gather(xbuf, [rv, c3[half] + 2])
            s = g0 * g0 + g1 * g1 + g2 * g2
            plsc.store_scatter(obuf, [ov, cout[4 + half]], _sqrt16(s))
        s = None
        for j in range(5):
            g = plsc.load_gather(xbuf, [rv, c5 + j])
            s = g * g if s is None else s + g * g
        plsc.store_scatter(obuf, [ov, cout[6]], _sqrt16(s))

    def in_copy(k, slot):
        row0 = (wid + k * _NW) * _CHUNK
        return pltpu.make_async_copy(
            x_hbm.at[pl.ds(row0, _CHUNK)],
            xbuf.at[pl.ds(slot * _CHUNK, _CHUNK)],
            insem.at[slot])

    def out_copy(k, slot):
        row0 = (wid + k * _NW) * _CHUNK
        return pltpu.make_async_copy(
            obuf.at[pl.ds(slot * _CHUNK, _CHUNK)],
            o_hbm.at[pl.ds(row0, _CHUNK)],
            outsem.at[slot])

    in_copy(0, 0).start()

    @pl.loop(0, nck)
    def _chunk(k):
        slot = lax.rem(k, 2)

        @pl.when(k + 1 < nck)
        def _prefetch():
            in_copy(k + 1, 1 - slot).start()

        in_copy(k, slot).wait()

        @pl.when(k >= 2)
        def _drain():
            out_copy(k - 2, slot).wait()

        base = slot * _CHUNK

        @plsc.parallel_loop(0, _CHUNK, unroll=4)
        def _row(r):
            row_norm(base + r, base + r)

        out_copy(k, slot).start()

    out_copy(nck - 2, lax.rem(nck - 2, 2)).wait()
    out_copy(nck - 1, lax.rem(nck - 1, 2)).wait()

    @pl.when(wid == _TAIL_W)
    def _tail():
        row0 = _NFULL * _CHUNK
        pltpu.sync_copy(x_hbm.at[pl.ds(row0, _TAIL)],
                        xbuf.at[pl.ds(0, _TAIL)])

        @plsc.parallel_loop(0, _TAIL, unroll=4)
        def _row(r):
            row_norm(r, r)

        pltpu.sync_copy(obuf.at[pl.ds(0, _TAIL)],
                        o_hbm.at[pl.ds(row0, _TAIL)])


def kernel(features):
    size = features.shape[:-1]
    x = features.reshape(-1, _DIM)
    mesh = plsc.VectorSubcoreMesh(core_axis_name="c", subcore_axis_name="s",
                                  num_cores=2, num_subcores=16)
    out = pl.kernel(
        _body,
        out_type=jax.ShapeDtypeStruct((_N, _NSEG), jnp.float32),
        mesh=mesh,
        scratch_types=[
            pltpu.VMEM((2 * _CHUNK, _DIM), jnp.float32),
            pltpu.VMEM((2 * _CHUNK, _NSEG), jnp.float32),
            pltpu.SemaphoreType.DMA((2,)),
            pltpu.SemaphoreType.DMA((2,)),
        ],
        compiler_params=pltpu.CompilerParams(needs_layout_passes=False),
    )(x)
    return out.reshape(size + (_NSEG,))
